# Initial kernel scaffold; baseline (speedup 1.0000x reference)
import jax
import jax.numpy as jnp
from jax.experimental import pallas as pl


def kernel(last_hidden_state, encoder_last_hidden_state, encoder_input_ids, encoder_attention_mask, emb_weight):
    raise NotImplementedError("write your pallas kernel here")


# speedup vs baseline: 2.2450x; 2.2450x over previous
import functools

import jax
import jax.numpy as jnp
from jax import lax
from jax.experimental import pallas as pl
from jax.experimental.pallas import tpu as pltpu
from jax.experimental.pallas import tpu_sc as plsc

B, L, N, H, V = 4, 512, 2048, 1024, 50265
_PTR_OFF = 10
_EOS_INPUT_ID = 2
_TGT_LO = 50255
_NEG_WORD = float(-1e32)
_NEG_PAD = float(-1e24)

_BN = B * N
_NC, _NS = 2, 16
_NW = _NC * _NS
_PER_W = _BN // _NW
_CHUNK = 64


def _sc_gather_body(table_hbm, idx_hbm, out_hbm, idx_v, rows_v, sem):
    wid = lax.axis_index("s") * _NC + lax.axis_index("c")
    base = wid * _PER_W
    for c in range(_PER_W // _CHUNK):
        off = base + c * _CHUNK
        pltpu.sync_copy(idx_hbm.at[pl.ds(off, _CHUNK)], idx_v)
        pltpu.async_copy(table_hbm.at[idx_v], rows_v, sem).wait()
        pltpu.sync_copy(rows_v, out_hbm.at[pl.ds(off, _CHUNK)])


@functools.cache
def _sc_gather():
    return pl.kernel(
        _sc_gather_body,
        mesh=plsc.VectorSubcoreMesh(core_axis_name="c", subcore_axis_name="s"),
        out_type=jax.ShapeDtypeStruct((_BN, H), jnp.float32),
        scratch_types=[
            pltpu.VMEM((_CHUNK,), jnp.int32),
            pltpu.VMEM((_CHUNK, H), jnp.float32),
            pltpu.SemaphoreType.DMA,
        ],
    )

_TN = 512


def _tc_body(lhs_ref, g_ref, s_ref, m_ref, wh_ref, word_ref, head_ref):
    j = pl.program_id(1)
    l = lhs_ref[0]
    keys = ((g_ref[0] + s_ref[0]) * 0.5).astype(jnp.bfloat16)
    a = lax.dot_general(l, keys, (((1,), (1,)), ((), ())),
                        preferred_element_type=jnp.float32)
    m = m_ref[0]
    word_ref[0] = jnp.where(m > 0, _NEG_WORD, a)

    @pl.when(j == 0)
    def _head():
        h = lax.dot_general(l, wh_ref[...], (((1,), (1,)), ((), ())),
                            preferred_element_type=jnp.float32)
        col = lax.broadcasted_iota(jnp.int32, (L, 16), 1)
        head_ref[0] = jnp.where(col == 0, _NEG_PAD, h)


_tc_call = pl.pallas_call(
    _tc_body,
    grid=(B, N // _TN),
    in_specs=[
        pl.BlockSpec((1, L, H), lambda b, j: (b, 0, 0)),
        pl.BlockSpec((1, _TN, H), lambda b, j: (b, j, 0)),
        pl.BlockSpec((1, _TN, H), lambda b, j: (b, j, 0)),
        pl.BlockSpec((1, 1, _TN), lambda b, j: (b, 0, j)),
        pl.BlockSpec((16, H), lambda b, j: (0, 0)),
    ],
    out_specs=[
        pl.BlockSpec((1, L, _TN), lambda b, j: (b, 0, j)),
        pl.BlockSpec((1, L, 16), lambda b, j: (b, 0, 0)),
    ],
    out_shape=[
        jax.ShapeDtypeStruct((B, L, N), jnp.float32),
        jax.ShapeDtypeStruct((B, L, 16), jnp.float32),
    ],
    compiler_params=pltpu.CompilerParams(
        dimension_semantics=("parallel", "arbitrary"),
    ),
)


def kernel(last_hidden_state, encoder_last_hidden_state, encoder_input_ids,
           encoder_attention_mask, emb_weight):
    ids = encoder_input_ids.astype(jnp.int32)
    gathered = _sc_gather()(emb_weight, ids.reshape(_BN)).reshape(B, N, H)

    eos_seen = jnp.cumsum((ids == _EOS_INPUT_ID).astype(jnp.int32), axis=1) >= 1
    maskf = ((encoder_attention_mask == 0) | eos_seen).astype(jnp.float32)
    maskf = maskf.reshape(B, 1, N)

    lhs_bf = last_hidden_state.astype(jnp.bfloat16)
    wh = jnp.concatenate(
        [lax.slice_in_dim(emb_weight, _TGT_LO, _TGT_LO + _PTR_OFF, axis=0),
         jnp.zeros((16 - _PTR_OFF, H), jnp.float32)], axis=0
    ).astype(jnp.bfloat16)

    word, head = _tc_call(lhs_bf, gathered, encoder_last_hidden_state,
                          maskf, wh)
    return jnp.concatenate([head[:, :, :_PTR_OFF], word], axis=-1)


# speedup vs baseline: 2.3050x; 1.0267x over previous
import functools

import jax
import jax.numpy as jnp
from jax import lax
from jax.experimental import pallas as pl
from jax.experimental.pallas import tpu as pltpu
from jax.experimental.pallas import tpu_sc as plsc

B, L, N, H, V = 4, 512, 2048, 1024, 50265
_PTR_OFF = 10
_EOS_INPUT_ID = 2
_TGT_LO = 50255
_NEG_WORD = float(-1e32)
_NEG_PAD = float(-1e24)

_BN = B * N
_NC, _NS = 2, 16
_NW = _NC * _NS
_PER_W = _BN // _NW
_CHUNK = 32


def _sc_gather_body(table_hbm, idx_hbm, out_hbm, idx_v, rows0, rows1,
                    gsem0, gsem1, wsem0, wsem1):
    wid = lax.axis_index("s") * _NC + lax.axis_index("c")
    base = wid * _PER_W
    pltpu.sync_copy(idx_hbm.at[pl.ds(base, _PER_W)], idx_v)
    bufs, gsems, wsems = (rows0, rows1), (gsem0, gsem1), (wsem0, wsem1)
    nch = _PER_W // _CHUNK
    hg, hw = [None] * nch, [None] * nch

    def start_gather(c):
        hg[c] = pltpu.async_copy(
            table_hbm.at[idx_v.at[pl.ds(c * _CHUNK, _CHUNK)]],
            bufs[c & 1], gsems[c & 1])

    def start_writeback(c):
        hw[c] = pltpu.async_copy(
            bufs[c & 1], out_hbm.at[pl.ds(base + c * _CHUNK, _CHUNK)],
            wsems[c & 1])

    start_gather(0)
    for c in range(nch):
        if c + 1 < nch:
            if c >= 1:
                hw[c - 1].wait()
            start_gather(c + 1)
        hg[c].wait()
        start_writeback(c)
    hw[nch - 2].wait()
    hw[nch - 1].wait()


@functools.cache
def _sc_gather():
    return pl.kernel(
        _sc_gather_body,
        mesh=plsc.VectorSubcoreMesh(core_axis_name="c", subcore_axis_name="s"),
        out_type=jax.ShapeDtypeStruct((_BN, H), jnp.float32),
        scratch_types=[
            pltpu.VMEM((_PER_W,), jnp.int32),
            pltpu.VMEM((_CHUNK, H), jnp.float32),
            pltpu.VMEM((_CHUNK, H), jnp.float32),
            pltpu.SemaphoreType.DMA,
            pltpu.SemaphoreType.DMA,
            pltpu.SemaphoreType.DMA,
            pltpu.SemaphoreType.DMA,
        ],
    )

_TN = 512


def _tc_body(lhs_ref, g_ref, s_ref, m_ref, wh_ref, word_ref, head_ref):
    j = pl.program_id(1)
    l = lhs_ref[0]
    keys = ((g_ref[0] + s_ref[0]) * 0.5).astype(jnp.bfloat16)
    a = lax.dot_general(l, keys, (((1,), (1,)), ((), ())),
                        preferred_element_type=jnp.float32)
    m = m_ref[0]
    word_ref[0] = jnp.where(m > 0, _NEG_WORD, a)

    @pl.when(j == 0)
    def _head():
        h = lax.dot_general(l, wh_ref[...], (((1,), (1,)), ((), ())),
                            preferred_element_type=jnp.float32)
        col = lax.broadcasted_iota(jnp.int32, (L, 16), 1)
        head_ref[0] = jnp.where(col == 0, _NEG_PAD, h)


_tc_call = pl.pallas_call(
    _tc_body,
    grid=(B, N // _TN),
    in_specs=[
        pl.BlockSpec((1, L, H), lambda b, j: (b, 0, 0)),
        pl.BlockSpec((1, _TN, H), lambda b, j: (b, j, 0)),
        pl.BlockSpec((1, _TN, H), lambda b, j: (b, j, 0)),
        pl.BlockSpec((1, 1, _TN), lambda b, j: (b, 0, j)),
        pl.BlockSpec((16, H), lambda b, j: (0, 0)),
    ],
    out_specs=[
        pl.BlockSpec((1, L, _TN), lambda b, j: (b, 0, j)),
        pl.BlockSpec((1, L, 16), lambda b, j: (b, 0, 0)),
    ],
    out_shape=[
        jax.ShapeDtypeStruct((B, L, N), jnp.float32),
        jax.ShapeDtypeStruct((B, L, 16), jnp.float32),
    ],
    compiler_params=pltpu.CompilerParams(
        dimension_semantics=("parallel", "arbitrary"),
    ),
)


def kernel(last_hidden_state, encoder_last_hidden_state, encoder_input_ids,
           encoder_attention_mask, emb_weight):
    ids = encoder_input_ids.astype(jnp.int32)
    gathered = _sc_gather()(emb_weight, ids.reshape(_BN)).reshape(B, N, H)

    eos_seen = jnp.cumsum((ids == _EOS_INPUT_ID).astype(jnp.int32), axis=1) >= 1
    maskf = ((encoder_attention_mask == 0) | eos_seen).astype(jnp.float32)
    maskf = maskf.reshape(B, 1, N)

    lhs_bf = last_hidden_state.astype(jnp.bfloat16)
    wh = jnp.concatenate(
        [lax.slice_in_dim(emb_weight, _TGT_LO, _TGT_LO + _PTR_OFF, axis=0),
         jnp.zeros((16 - _PTR_OFF, H), jnp.float32)], axis=0
    ).astype(jnp.bfloat16)

    word, head = _tc_call(lhs_bf, gathered, encoder_last_hidden_state,
                          maskf, wh)
    return jnp.concatenate([head[:, :, :_PTR_OFF], word], axis=-1)


# speedup vs baseline: 2.4602x; 1.0673x over previous
import functools

import jax
import jax.numpy as jnp
from jax import lax
from jax.experimental import pallas as pl
from jax.experimental.pallas import tpu as pltpu
from jax.experimental.pallas import tpu_sc as plsc

B, L, N, H, V = 4, 512, 2048, 1024, 50265
_PTR_OFF = 10
_EOS_INPUT_ID = 2
_TGT_LO = 50255
_NEG_WORD = float(-1e32)
_NEG_PAD = float(-1e24)

_BN = B * N
_NC, _NS = 2, 16
_NW = _NC * _NS
_PER_W = _BN // _NW
_CHUNK = 32


def _sc_gather_body(table_hbm, idx_hbm, out_hbm, idx_v, rows0, rows1,
                    gsem0, gsem1, wsem0, wsem1):
    wid = lax.axis_index("s") * _NC + lax.axis_index("c")
    base = wid * _PER_W
    pltpu.sync_copy(idx_hbm.at[pl.ds(base, _PER_W)], idx_v)
    bufs, gsems, wsems = (rows0, rows1), (gsem0, gsem1), (wsem0, wsem1)
    nch = _PER_W // _CHUNK
    hg, hw = [None] * nch, [None] * nch

    def start_gather(c):
        hg[c] = pltpu.async_copy(
            table_hbm.at[idx_v.at[pl.ds(c * _CHUNK, _CHUNK)]],
            bufs[c & 1], gsems[c & 1])

    def start_writeback(c):
        hw[c] = pltpu.async_copy(
            bufs[c & 1], out_hbm.at[pl.ds(base + c * _CHUNK, _CHUNK)],
            wsems[c & 1])

    start_gather(0)
    for c in range(nch):
        if c + 1 < nch:
            if c >= 1:
                hw[c - 1].wait()
            start_gather(c + 1)
        hg[c].wait()
        start_writeback(c)
    hw[nch - 2].wait()
    hw[nch - 1].wait()


@functools.cache
def _sc_gather():
    return pl.kernel(
        _sc_gather_body,
        mesh=plsc.VectorSubcoreMesh(core_axis_name="c", subcore_axis_name="s"),
        out_type=jax.ShapeDtypeStruct((_BN, H), jnp.float32),
        scratch_types=[
            pltpu.VMEM((_PER_W,), jnp.int32),
            pltpu.VMEM((_CHUNK, H), jnp.float32),
            pltpu.VMEM((_CHUNK, H), jnp.float32),
            pltpu.SemaphoreType.DMA,
            pltpu.SemaphoreType.DMA,
            pltpu.SemaphoreType.DMA,
            pltpu.SemaphoreType.DMA,
        ],
    )

_TN = 512


def _tc_body(lhs_ref, g_ref, s_ref, m_ref, wh_ref, out_ref):
    j = pl.program_id(1)
    l = lhs_ref[0]
    keys = ((g_ref[0] + s_ref[0]) * 0.5).astype(jnp.bfloat16)
    a = lax.dot_general(l, keys, (((1,), (1,)), ((), ())),
                        preferred_element_type=jnp.float32)
    m = m_ref[0]
    masked = jnp.where(m > 0, _NEG_WORD, a)

    @pl.when(j == 0)
    def _head():
        h = lax.dot_general(l, wh_ref[...], (((1,), (1,)), ((), ())),
                            preferred_element_type=jnp.float32)
        col = lax.broadcasted_iota(jnp.int32, (L, 16), 1)
        out_ref[0, :, pl.ds(0, 16)] = jnp.where(col == 0, _NEG_PAD, h)

    for k in range(N // _TN):
        @pl.when(j == k)
        def _store(k=k):
            out_ref[0, :, pl.ds(_PTR_OFF + k * _TN, _TN)] = masked


_tc_call = pl.pallas_call(
    _tc_body,
    grid=(B, N // _TN),
    in_specs=[
        pl.BlockSpec((1, L, H), lambda b, j: (b, 0, 0)),
        pl.BlockSpec((1, _TN, H), lambda b, j: (b, j, 0)),
        pl.BlockSpec((1, _TN, H), lambda b, j: (b, j, 0)),
        pl.BlockSpec((1, 1, _TN), lambda b, j: (b, 0, j)),
        pl.BlockSpec((16, H), lambda b, j: (0, 0)),
    ],
    out_specs=pl.BlockSpec((1, L, _PTR_OFF + N), lambda b, j: (b, 0, 0)),
    out_shape=jax.ShapeDtypeStruct((B, L, _PTR_OFF + N), jnp.float32),
    compiler_params=pltpu.CompilerParams(
        dimension_semantics=("parallel", "arbitrary"),
    ),
)


def kernel(last_hidden_state, encoder_last_hidden_state, encoder_input_ids,
           encoder_attention_mask, emb_weight):
    ids = encoder_input_ids.astype(jnp.int32)
    gathered = _sc_gather()(emb_weight, ids.reshape(_BN)).reshape(B, N, H)

    eos_seen = jnp.cumsum((ids == _EOS_INPUT_ID).astype(jnp.int32), axis=1) >= 1
    maskf = ((encoder_attention_mask == 0) | eos_seen).astype(jnp.float32)
    maskf = maskf.reshape(B, 1, N)

    lhs_bf = last_hidden_state.astype(jnp.bfloat16)
    wh = jnp.concatenate(
        [lax.slice_in_dim(emb_weight, _TGT_LO, _TGT_LO + _PTR_OFF, axis=0),
         jnp.zeros((16 - _PTR_OFF, H), jnp.float32)], axis=0
    ).astype(jnp.bfloat16)

    return _tc_call(lhs_bf, gathered, encoder_last_hidden_state, maskf, wh)
